# baseline (speedup 1.0000x reference)
import jax
import jax.numpy as jnp
from jax import lax
from jax.experimental import pallas as pl
from jax.experimental.pallas import tpu as pltpu
from jax.experimental.pallas import tpu_sc as plsc

_NC = 2
_NS = 16
_NW = _NC * _NS

_U_SHARD, _U_CHUNK = 31248, 504
_I_SHARD, _I_CHUNK = 3120, 312


def _stream_copy(src, dst, base, n, chunk, bufs, isems, osems):
    def in_cp(k, b):
        return pltpu.make_async_copy(
            src.at[pl.ds(base + k * chunk, chunk), :],
            bufs[b].at[pl.ds(0, chunk), :], isems[b])

    def out_cp(k, b):
        return pltpu.make_async_copy(
            bufs[b].at[pl.ds(0, chunk), :],
            dst.at[pl.ds(base + k * chunk, chunk), :], osems[b])

    in_cp(0, 0).start()

    @pl.loop(0, n, step=2)
    def _pair(g):
        for b in range(2):
            k = g + b
            in_cp(k, b).wait()
            out_cp(k, b).start()
            nxt = k + 1

            @pl.when(nxt < n)
            def _start_next():
                @pl.when(k >= 1)
                def _drain_prev():
                    out_cp(k - 1, 1 - b).wait()

                in_cp(nxt, 1 - b).start()

    out_cp(n - 2, 0).wait()
    out_cp(n - 1, 1).wait()


def _tail_copy(src, dst, base, rows, buf, isem, osem):
    pltpu.make_async_copy(
        src.at[pl.ds(base, rows), :], buf.at[pl.ds(0, rows), :], isem).start()
    pltpu.make_async_copy(
        src.at[pl.ds(base, rows), :], buf.at[pl.ds(0, rows), :], isem).wait()
    pltpu.make_async_copy(
        buf.at[pl.ds(0, rows), :], dst.at[pl.ds(base, rows), :], osem).start()
    pltpu.make_async_copy(
        buf.at[pl.ds(0, rows), :], dst.at[pl.ds(base, rows), :], osem).wait()


def _copy_body(u_hbm, i_hbm, uo_hbm, io_hbm, buf0, buf1, is0, is1, os0, os1):
    wid = lax.axis_index("s") * _NC + lax.axis_index("c")
    bufs, isems, osems = (buf0, buf1), (is0, is1), (os0, os1)

    _stream_copy(u_hbm, uo_hbm, wid * _U_SHARD, _U_SHARD // _U_CHUNK,
                 _U_CHUNK, bufs, isems, osems)
    _stream_copy(i_hbm, io_hbm, wid * _I_SHARD, _I_SHARD // _I_CHUNK,
                 _I_CHUNK, bufs, isems, osems)

    n_users, n_items = u_hbm.shape[0], i_hbm.shape[0]

    @pl.when(wid == 0)
    def _tails():
        _tail_copy(u_hbm, uo_hbm, _NW * _U_SHARD, n_users - _NW * _U_SHARD,
                   buf0, is0, os0)
        _tail_copy(i_hbm, io_hbm, _NW * _I_SHARD, n_items - _NW * _I_SHARD,
                   buf0, is0, os0)


def kernel(user_embeds, item_embeds, adj):
    d = user_embeds.shape[1]
    sc_copy = pl.kernel(
        _copy_body,
        out_type=(
            jax.ShapeDtypeStruct(user_embeds.shape, user_embeds.dtype),
            jax.ShapeDtypeStruct(item_embeds.shape, item_embeds.dtype),
        ),
        mesh=plsc.VectorSubcoreMesh(core_axis_name="c", subcore_axis_name="s"),
        scratch_types=[
            pltpu.VMEM((_U_CHUNK, d), jnp.float32),
            pltpu.VMEM((_U_CHUNK, d), jnp.float32),
            pltpu.SemaphoreType.DMA,
            pltpu.SemaphoreType.DMA,
            pltpu.SemaphoreType.DMA,
            pltpu.SemaphoreType.DMA,
        ],
        compiler_params=pltpu.CompilerParams(use_tc_tiling_on_sc=True),
    )
    return sc_copy(user_embeds, item_embeds)


# speedup vs baseline: 5.3113x; 5.3113x over previous
import jax
import jax.numpy as jnp
from jax import lax
from jax.experimental import pallas as pl
from jax.experimental.pallas import tpu as pltpu
from jax.experimental.pallas import tpu_sc as plsc

_NC = 2
_NS = 16
_NW = _NC * _NS

_U_SHARD, _U_CHUNK = 31232, 512
_I_SHARD, _I_CHUNK = 3072, 512


def _stream_copy(src, dst, base, n, chunk, bufs, isems, osems):
    def in_cp(k, b):
        return pltpu.make_async_copy(
            src.at[:, pl.ds(base + k * chunk, chunk)],
            bufs[b].at[:, pl.ds(0, chunk)], isems[b])

    def out_cp(k, b):
        return pltpu.make_async_copy(
            bufs[b].at[:, pl.ds(0, chunk)],
            dst.at[:, pl.ds(base + k * chunk, chunk)], osems[b])

    in_cp(0, 0).start()

    @pl.loop(0, 2 * (n // 2), step=2)
    def _pair(g):
        for b in range(2):
            k = g + b
            in_cp(k, b).wait()
            out_cp(k, b).start()
            nxt = k + 1

            @pl.when(nxt < n)
            def _start_next():
                @pl.when(k >= 1)
                def _drain_prev():
                    out_cp(k - 1, 1 - b).wait()

                in_cp(nxt, 1 - b).start()

    if n % 2:
        in_cp(n - 1, (n - 1) % 2).wait()
        out_cp(n - 1, (n - 1) % 2).start()
    out_cp(n - 2, (n - 2) % 2).wait()
    out_cp(n - 1, (n - 1) % 2).wait()


def _tail_copy(src, dst, col0, cols, buf, isem, osem):
    pltpu.make_async_copy(
        src.at[:, pl.ds(col0, cols)], buf.at[:, pl.ds(0, cols)], isem).start()
    pltpu.make_async_copy(
        src.at[:, pl.ds(col0, cols)], buf.at[:, pl.ds(0, cols)], isem).wait()
    pltpu.make_async_copy(
        buf.at[:, pl.ds(0, cols)], dst.at[:, pl.ds(col0, cols)], osem).start()
    pltpu.make_async_copy(
        buf.at[:, pl.ds(0, cols)], dst.at[:, pl.ds(col0, cols)], osem).wait()


def _copy_body(u_hbm, i_hbm, uo_hbm, io_hbm, buf0, buf1, is0, is1, os0, os1):
    wid = lax.axis_index("s") * _NC + lax.axis_index("c")
    bufs, isems, osems = (buf0, buf1), (is0, is1), (os0, os1)

    _stream_copy(u_hbm, uo_hbm, wid * _U_SHARD, _U_SHARD // _U_CHUNK,
                 _U_CHUNK, bufs, isems, osems)
    _stream_copy(i_hbm, io_hbm, wid * _I_SHARD, _I_SHARD // _I_CHUNK,
                 _I_CHUNK, bufs, isems, osems)

    n_u, n_i = u_hbm.shape[1], i_hbm.shape[1]
    u_base, i_base = _NW * _U_SHARD, _NW * _I_SHARD
    u_full = (n_u - u_base) // 128
    i_full = (n_i - i_base) // 128

    @pl.when(wid < u_full)
    def _u_tail():
        _tail_copy(u_hbm, uo_hbm, u_base + 128 * wid, 128, buf0, is0, os0)

    @pl.when(wid < i_full)
    def _i_tail():
        _tail_copy(i_hbm, io_hbm, i_base + 128 * wid, 128, buf1, is1, os1)


def kernel(user_embeds, item_embeds, adj):
    d = user_embeds.shape[1]
    u_t = user_embeds.T
    i_t = item_embeds.T
    sc_copy = pl.kernel(
        _copy_body,
        out_type=(
            jax.ShapeDtypeStruct(u_t.shape, u_t.dtype),
            jax.ShapeDtypeStruct(i_t.shape, i_t.dtype),
        ),
        mesh=plsc.VectorSubcoreMesh(core_axis_name="c", subcore_axis_name="s"),
        scratch_types=[
            pltpu.VMEM((d, _U_CHUNK), jnp.float32),
            pltpu.VMEM((d, _U_CHUNK), jnp.float32),
            pltpu.SemaphoreType.DMA,
            pltpu.SemaphoreType.DMA,
            pltpu.SemaphoreType.DMA,
            pltpu.SemaphoreType.DMA,
        ],
        compiler_params=pltpu.CompilerParams(use_tc_tiling_on_sc=True),
    )
    u_out, i_out = sc_copy(u_t, i_t)
    n_u, n_i = u_t.shape[1], i_t.shape[1]
    u_a, i_a = n_u - n_u % 128, n_i - n_i % 128
    u_out = jax.lax.dynamic_update_slice(u_out, u_t[:, u_a:], (0, u_a))
    i_out = jax.lax.dynamic_update_slice(i_out, i_t[:, i_a:], (0, i_a))
    return (u_out.T, i_out.T)


# speedup vs baseline: 5.3185x; 1.0014x over previous
import jax
import jax.numpy as jnp
from jax import lax
from jax.experimental import pallas as pl
from jax.experimental.pallas import tpu as pltpu
from jax.experimental.pallas import tpu_sc as plsc

_NC = 2
_NS = 16
_NW = _NC * _NS

_NBUF = 4
_U_SHARD, _U_CHUNK = 31232, 256
_I_SHARD, _I_CHUNK = 3072, 256


def _stream_copy(src, dst, base, n, chunk, bufs, isems, osems):
    nb = len(bufs)

    def in_cp(k, b):
        return pltpu.make_async_copy(
            src.at[:, pl.ds(base + k * chunk, chunk)],
            bufs[b].at[:, pl.ds(0, chunk)], isems[b])

    def out_cp(k, b):
        return pltpu.make_async_copy(
            bufs[b].at[:, pl.ds(0, chunk)],
            dst.at[:, pl.ds(base + k * chunk, chunk)], osems[b])

    for j in range(min(nb - 1, n)):
        in_cp(j, j).start()

    main = n - n % nb

    @pl.loop(0, main, step=nb)
    def _group(g):
        for b in range(nb):
            k = g + b
            in_cp(k, b).wait()
            out_cp(k, b).start()
            nxt = k + nb - 1

            @pl.when(nxt < n)
            def _start_next():
                @pl.when(k >= 1)
                def _drain_prev():
                    out_cp(k - 1, (b - 1) % nb).wait()

                in_cp(nxt, (b - 1) % nb).start()

    for k in range(main, n):
        in_cp(k, k % nb).wait()
        out_cp(k, k % nb).start()
    for k in range(max(0, n - nb), n):
        out_cp(k, k % nb).wait()


def _tail_copy(src, dst, col0, cols, buf, isem, osem):
    pltpu.make_async_copy(
        src.at[:, pl.ds(col0, cols)], buf.at[:, pl.ds(0, cols)], isem).start()
    pltpu.make_async_copy(
        src.at[:, pl.ds(col0, cols)], buf.at[:, pl.ds(0, cols)], isem).wait()
    pltpu.make_async_copy(
        buf.at[:, pl.ds(0, cols)], dst.at[:, pl.ds(col0, cols)], osem).start()
    pltpu.make_async_copy(
        buf.at[:, pl.ds(0, cols)], dst.at[:, pl.ds(col0, cols)], osem).wait()


def _copy_body(u_hbm, i_hbm, uo_hbm, io_hbm, b0, b1, b2, b3,
               is0, is1, is2, is3, os0, os1, os2, os3):
    wid = lax.axis_index("s") * _NC + lax.axis_index("c")
    bufs = (b0, b1, b2, b3)
    isems, osems = (is0, is1, is2, is3), (os0, os1, os2, os3)

    _stream_copy(u_hbm, uo_hbm, wid * _U_SHARD, _U_SHARD // _U_CHUNK,
                 _U_CHUNK, bufs, isems, osems)
    _stream_copy(i_hbm, io_hbm, wid * _I_SHARD, _I_SHARD // _I_CHUNK,
                 _I_CHUNK, bufs, isems, osems)

    n_u, n_i = u_hbm.shape[1], i_hbm.shape[1]
    u_base, i_base = _NW * _U_SHARD, _NW * _I_SHARD
    u_full = (n_u - u_base) // 128
    i_full = (n_i - i_base) // 128

    @pl.when(wid < u_full)
    def _u_tail():
        _tail_copy(u_hbm, uo_hbm, u_base + 128 * wid, 128, b0, is0, os0)

    @pl.when(wid < i_full)
    def _i_tail():
        _tail_copy(i_hbm, io_hbm, i_base + 128 * wid, 128, b1, is1, os1)


def kernel(user_embeds, item_embeds, adj):
    d = user_embeds.shape[1]
    u_t = user_embeds.T
    i_t = item_embeds.T
    sc_copy = pl.kernel(
        _copy_body,
        out_type=(
            jax.ShapeDtypeStruct(u_t.shape, u_t.dtype),
            jax.ShapeDtypeStruct(i_t.shape, i_t.dtype),
        ),
        mesh=plsc.VectorSubcoreMesh(core_axis_name="c", subcore_axis_name="s"),
        scratch_types=(
            [pltpu.VMEM((d, _U_CHUNK), jnp.float32)] * _NBUF
            + [pltpu.SemaphoreType.DMA] * (2 * _NBUF)
        ),
        compiler_params=pltpu.CompilerParams(use_tc_tiling_on_sc=True),
    )
    u_out, i_out = sc_copy(u_t, i_t)
    n_u, n_i = u_t.shape[1], i_t.shape[1]
    u_a, i_a = n_u - n_u % 128, n_i - n_i % 128
    u_out = jax.lax.dynamic_update_slice(u_out, u_t[:, u_a:], (0, u_a))
    i_out = jax.lax.dynamic_update_slice(i_out, i_t[:, i_a:], (0, i_a))
    return (u_out.T, i_out.T)


# speedup vs baseline: 5.4613x; 1.0269x over previous
import jax
import jax.numpy as jnp
from jax import lax
from jax.experimental import pallas as pl
from jax.experimental.pallas import tpu as pltpu
from jax.experimental.pallas import tpu_sc as plsc

_NC = 2
_NS = 16
_NW = _NC * _NS
_NBUF = 4

_U_SHARD, _U_CHUNK = 31232, 256
_I_BLOCK = 4096


def _stream_copy(src, dst, base, n, chunk, bufs, isems, osems):
    nb = len(bufs)

    def in_cp(k, b):
        return pltpu.make_async_copy(
            src.at[:, pl.ds(base + k * chunk, chunk)],
            bufs[b].at[:, pl.ds(0, chunk)], isems[b])

    def out_cp(k, b):
        return pltpu.make_async_copy(
            bufs[b].at[:, pl.ds(0, chunk)],
            dst.at[:, pl.ds(base + k * chunk, chunk)], osems[b])

    for j in range(min(nb - 1, n)):
        in_cp(j, j).start()

    main = n - n % nb

    @pl.loop(0, main, step=nb)
    def _group(g):
        for b in range(nb):
            k = g + b
            in_cp(k, b).wait()
            out_cp(k, b).start()
            nxt = k + nb - 1

            @pl.when(nxt < n)
            def _start_next():
                @pl.when(k >= 1)
                def _drain_prev():
                    out_cp(k - 1, (b - 1) % nb).wait()

                in_cp(nxt, (b - 1) % nb).start()

    for k in range(main, n):
        in_cp(k, k % nb).wait()
        out_cp(k, k % nb).start()
    for k in range(max(0, n - nb), n):
        out_cp(k, k % nb).wait()


def _tail_copy(src, dst, col0, cols, buf, isem, osem):
    pltpu.make_async_copy(
        src.at[:, pl.ds(col0, cols)], buf.at[:, pl.ds(0, cols)], isem).start()
    pltpu.make_async_copy(
        src.at[:, pl.ds(col0, cols)], buf.at[:, pl.ds(0, cols)], isem).wait()
    pltpu.make_async_copy(
        buf.at[:, pl.ds(0, cols)], dst.at[:, pl.ds(col0, cols)], osem).start()
    pltpu.make_async_copy(
        buf.at[:, pl.ds(0, cols)], dst.at[:, pl.ds(col0, cols)], osem).wait()


def _sc_body(u_hbm, uo_hbm, b0, b1, b2, b3,
             is0, is1, is2, is3, os0, os1, os2, os3):
    wid = lax.axis_index("s") * _NC + lax.axis_index("c")
    bufs = (b0, b1, b2, b3)
    isems, osems = (is0, is1, is2, is3), (os0, os1, os2, os3)

    _stream_copy(u_hbm, uo_hbm, wid * _U_SHARD, _U_SHARD // _U_CHUNK,
                 _U_CHUNK, bufs, isems, osems)

    u_base = _NW * _U_SHARD
    u_full = (u_hbm.shape[1] - u_base) // 128

    @pl.when(wid < u_full)
    def _u_tail():
        _tail_copy(u_hbm, uo_hbm, u_base + 128 * wid, 128, b0, is0, os0)


def _tc_body(src, dst):
    dst[...] = src[...]


def kernel(user_embeds, item_embeds, adj):
    d = user_embeds.shape[1]
    u_t = user_embeds.T
    i_t = item_embeds.T

    sc_copy = pl.kernel(
        _sc_body,
        out_type=jax.ShapeDtypeStruct(u_t.shape, u_t.dtype),
        mesh=plsc.VectorSubcoreMesh(core_axis_name="c", subcore_axis_name="s"),
        scratch_types=(
            [pltpu.VMEM((d, _U_CHUNK), jnp.float32)] * _NBUF
            + [pltpu.SemaphoreType.DMA] * (2 * _NBUF)
        ),
        compiler_params=pltpu.CompilerParams(use_tc_tiling_on_sc=True),
    )
    u_out = sc_copy(u_t)

    n_i = i_t.shape[1]
    i_out = pl.pallas_call(
        _tc_body,
        grid=(pl.cdiv(n_i, _I_BLOCK),),
        in_specs=[pl.BlockSpec((d, _I_BLOCK), lambda g: (0, g))],
        out_specs=pl.BlockSpec((d, _I_BLOCK), lambda g: (0, g)),
        out_shape=jax.ShapeDtypeStruct(i_t.shape, i_t.dtype),
    )(i_t)

    n_u = u_t.shape[1]
    u_a = n_u - n_u % 128
    u_out = jax.lax.dynamic_update_slice(u_out, u_t[:, u_a:], (0, u_a))
    return (u_out.T, i_out.T)


# speedup vs baseline: 5.4986x; 1.0068x over previous
import jax
import jax.numpy as jnp
from jax import lax
from jax.experimental import pallas as pl
from jax.experimental.pallas import tpu as pltpu
from jax.experimental.pallas import tpu_sc as plsc

_NC = 2
_NS = 16
_NW = _NC * _NS

_U_QUART, _U_CHUNK = 249856, 7808
_I_BLOCK = 4096


def _stream_copy(src, dst, row0, rows, base, n, chunk, bufs, isems, osems):
    nb = len(bufs)

    def in_cp(k, b):
        return pltpu.make_async_copy(
            src.at[pl.ds(row0, rows), pl.ds(base + k * chunk, chunk)],
            bufs[b], isems[b])

    def out_cp(k, b):
        return pltpu.make_async_copy(
            bufs[b],
            dst.at[pl.ds(row0, rows), pl.ds(base + k * chunk, chunk)],
            osems[b])

    for j in range(min(nb - 1, n)):
        in_cp(j, j).start()

    main = n - n % nb

    @pl.loop(0, main, step=nb)
    def _group(g):
        for b in range(nb):
            k = g + b
            in_cp(k, b).wait()
            out_cp(k, b).start()
            nxt = k + nb - 1

            @pl.when(nxt < n)
            def _start_next():
                @pl.when(k >= 1)
                def _drain_prev():
                    out_cp(k - 1, (b - 1) % nb).wait()

                in_cp(nxt, (b - 1) % nb).start()

    for k in range(main, n):
        in_cp(k, k % nb).wait()
        out_cp(k, k % nb).start()
    for k in range(max(0, n - nb), n):
        out_cp(k, k % nb).wait()


def _sc_body(u_hbm, uo_hbm, b0, b1, is0, is1, os0, os1):
    wid = lax.axis_index("s") * _NC + lax.axis_index("c")
    band, quart = wid % 8, wid // 8

    _stream_copy(u_hbm, uo_hbm, 8 * band, 8, quart * _U_QUART,
                 _U_QUART // _U_CHUNK, _U_CHUNK,
                 (b0, b1), (is0, is1), (os0, os1))

    u_base = 4 * _U_QUART
    t_row0, t_col0 = 8 * (wid // 4), u_base + 128 * (wid % 4)
    t_src = u_hbm.at[pl.ds(t_row0, 8), pl.ds(t_col0, 128)]
    t_dst = uo_hbm.at[pl.ds(t_row0, 8), pl.ds(t_col0, 128)]
    t_buf = b0.at[:, pl.ds(0, 128)]
    pltpu.make_async_copy(t_src, t_buf, is0).start()
    pltpu.make_async_copy(t_src, t_buf, is0).wait()
    pltpu.make_async_copy(t_buf, t_dst, os0).start()
    pltpu.make_async_copy(t_buf, t_dst, os0).wait()


def _tc_body(src, dst):
    dst[...] = src[...]


def kernel(user_embeds, item_embeds, adj):
    d = user_embeds.shape[1]
    u_t = user_embeds.T
    i_t = item_embeds.T

    sc_copy = pl.kernel(
        _sc_body,
        out_type=jax.ShapeDtypeStruct(u_t.shape, u_t.dtype),
        mesh=plsc.VectorSubcoreMesh(core_axis_name="c", subcore_axis_name="s"),
        scratch_types=(
            [pltpu.VMEM((8, _U_CHUNK), jnp.float32)] * 2
            + [pltpu.SemaphoreType.DMA] * 4
        ),
        compiler_params=pltpu.CompilerParams(use_tc_tiling_on_sc=True),
    )
    u_out = sc_copy(u_t)

    n_i = i_t.shape[1]
    i_out = pl.pallas_call(
        _tc_body,
        grid=(pl.cdiv(n_i, _I_BLOCK),),
        in_specs=[pl.BlockSpec((d, _I_BLOCK), lambda g: (0, g))],
        out_specs=pl.BlockSpec((d, _I_BLOCK), lambda g: (0, g)),
        out_shape=jax.ShapeDtypeStruct(i_t.shape, i_t.dtype),
    )(i_t)

    n_u = u_t.shape[1]
    u_a = n_u - n_u % 128
    u_out = jax.lax.dynamic_update_slice(u_out, u_t[:, u_a:], (0, u_a))
    return (u_out.T, i_out.T)
